# Initial kernel scaffold; baseline (speedup 1.0000x reference)
import jax
import jax.numpy as jnp
from jax.experimental import pallas as pl


def kernel(obs, W_obs, db_embeddings, top_k):
    raise NotImplementedError("write your pallas kernel here")


# speedup vs baseline: 2.1805x; 2.1805x over previous
import functools

import jax
import jax.numpy as jnp
from jax import lax
from jax.experimental import pallas as pl
from jax.experimental.pallas import tpu as pltpu
from jax.experimental.pallas import tpu_sc as plsc

B = 1024
OBS_D = 512
E = 128
DB = 100000
K = 4

BT = 512
DT = 2048
NBT = B // BT
NDT = (DB + DT - 1) // DT

_IMAX = 2147483647

_NC = 2
_NS = 16
_NW = _NC * _NS
_BPW = (B * K) // _NW


def _topk_body(obs_ref, w_ref, db_ref, vals_ref, idx_ref, q_s):
    dt = pl.program_id(1)

    @pl.when(dt == 0)
    def _init():
        q_s[...] = jnp.dot(
            obs_ref[...], w_ref[...],
            preferred_element_type=jnp.float32)
        vals_ref[...] = jnp.full(vals_ref.shape, -jnp.inf, jnp.float32)
        idx_ref[...] = jnp.full(idx_ref.shape, _IMAX, jnp.int32)

    sim = lax.dot_general(
        q_s[...], db_ref[...],
        (((1,), (1,)), ((), ())),
        preferred_element_type=jnp.float32)
    col = lax.broadcasted_iota(jnp.int32, (BT, DT), 1) + dt * DT
    sim = jnp.where(col < DB, sim, -jnp.inf)

    tv, ti = [], []
    for j in range(K):
        m = jnp.max(sim, axis=1, keepdims=True)
        p = jnp.min(jnp.where(sim == m, col, _IMAX), axis=1, keepdims=True)
        tv.append(m)
        ti.append(p)
        if j < K - 1:
            sim = jnp.where(col == p, -jnp.inf, sim)

    cvals = jnp.concatenate([vals_ref[...]] + tv, axis=1)
    cidx = jnp.concatenate([idx_ref[...]] + ti, axis=1)
    nv, ni = [], []
    for j in range(K):
        m = jnp.max(cvals, axis=1, keepdims=True)
        s = jnp.min(jnp.where(cvals == m, cidx, _IMAX), axis=1, keepdims=True)
        nv.append(m)
        ni.append(s)
        if j < K - 1:
            cvals = jnp.where(cidx == s, -jnp.inf, cvals)
    vals_ref[...] = jnp.concatenate(nv, axis=1)
    idx_ref[...] = jnp.concatenate(ni, axis=1)


def _topk_call(obs, w, db):
    return pl.pallas_call(
        _topk_body,
        grid=(NBT, NDT),
        in_specs=[
            pl.BlockSpec((BT, OBS_D), lambda bt, dt: (bt, 0)),
            pl.BlockSpec((OBS_D, E), lambda bt, dt: (0, 0)),
            pl.BlockSpec((DT, E), lambda bt, dt: (dt, 0)),
        ],
        out_specs=[
            pl.BlockSpec((BT, K), lambda bt, dt: (bt, 0)),
            pl.BlockSpec((BT, K), lambda bt, dt: (bt, 0)),
        ],
        out_shape=[
            jax.ShapeDtypeStruct((B, K), jnp.float32),
            jax.ShapeDtypeStruct((B, K), jnp.int32),
        ],
        scratch_shapes=[pltpu.VMEM((BT, E), jnp.float32)],
        compiler_params=pltpu.CompilerParams(
            dimension_semantics=("arbitrary", "arbitrary")),
    )(obs, w, db)


def _gather_body(db_hbm, idx_hbm, out_hbm, idx_v, rows_v, sem):
    wid = lax.axis_index("s") * _NC + lax.axis_index("c")
    base = wid * _BPW
    pltpu.sync_copy(idx_hbm.at[pl.ds(base, _BPW)], idx_v)
    pltpu.async_copy(db_hbm.at[idx_v], rows_v, sem).wait()
    pltpu.sync_copy(rows_v, out_hbm.at[pl.ds(base, _BPW)])


@functools.lru_cache(maxsize=1)
def _gather_call():
    return pl.kernel(
        _gather_body,
        mesh=plsc.VectorSubcoreMesh(core_axis_name="c", subcore_axis_name="s"),
        out_type=jax.ShapeDtypeStruct((B * K, E), jnp.float32),
        scratch_types=[
            pltpu.VMEM((_BPW,), jnp.int32),
            pltpu.VMEM((_BPW, E), jnp.float32),
            pltpu.SemaphoreType.DMA,
        ],
    )


def kernel(obs, W_obs, db_embeddings, top_k):
    del top_k
    _, idx = _topk_call(obs, W_obs, db_embeddings)
    rows = _gather_call()(db_embeddings, idx.reshape(B * K))
    memory = rows.reshape(B, K * E)
    return memory, idx


# speedup vs baseline: 2.9629x; 1.3588x over previous
import functools

import jax
import jax.numpy as jnp
from jax import lax
from jax.experimental import pallas as pl
from jax.experimental.pallas import tpu as pltpu
from jax.experimental.pallas import tpu_sc as plsc

B = 1024
OBS_D = 512
E = 128
DB = 100000
K = 4

BT = 512
DT = 4096
NBT = B // BT
NDT = (DB + DT - 1) // DT

_NC = 2
_NS = 16
_NW = _NC * _NS
_BPW = (B * K) // _NW


def _topk_body(obs_ref, w_ref, db_ref, idx_ref, q_s, rv_s, ri_s):
    dt = pl.program_id(1)

    @pl.when(dt == 0)
    def _init():
        q_s[...] = jnp.dot(
            obs_ref[...], w_ref[...],
            preferred_element_type=jnp.float32)
        rv_s[...] = jnp.full(rv_s.shape, -jnp.inf, jnp.float32)
        ri_s[...] = jnp.full(ri_s.shape, jnp.inf, jnp.float32)

    sim = lax.dot_general(
        q_s[...], db_ref[...],
        (((1,), (1,)), ((), ())),
        preferred_element_type=jnp.float32)
    col = (lax.broadcasted_iota(jnp.int32, (BT, DT), 1)
           + dt * DT).astype(jnp.float32)
    sim = jnp.where(col < float(DB), sim, -jnp.inf)

    tv, ti = [], []
    for j in range(K):
        m = jnp.max(sim, axis=1, keepdims=True)
        p = jnp.min(jnp.where(sim == m, col, jnp.inf), axis=1, keepdims=True)
        tv.append(m)
        ti.append(p)
        if j < K - 1:
            sim = jnp.where(col == p, -jnp.inf, sim)

    cvals = jnp.concatenate([rv_s[...]] + tv, axis=1)
    cidx = jnp.concatenate([ri_s[...]] + ti, axis=1)
    nv, ni = [], []
    for j in range(K):
        m = jnp.max(cvals, axis=1, keepdims=True)
        s = jnp.min(jnp.where(cvals == m, cidx, jnp.inf), axis=1, keepdims=True)
        nv.append(m)
        ni.append(s)
        if j < K - 1:
            cvals = jnp.where(cidx == s, -jnp.inf, cvals)
    rv_s[...] = jnp.concatenate(nv, axis=1)
    ri_s[...] = jnp.concatenate(ni, axis=1)

    @pl.when(dt == NDT - 1)
    def _finish():
        idx_ref[...] = ri_s[...].astype(jnp.int32)


def _topk_call(obs, w, db):
    return pl.pallas_call(
        _topk_body,
        grid=(NBT, NDT),
        in_specs=[
            pl.BlockSpec((BT, OBS_D), lambda bt, dt: (bt, 0)),
            pl.BlockSpec((OBS_D, E), lambda bt, dt: (0, 0)),
            pl.BlockSpec((DT, E), lambda bt, dt: (dt, 0)),
        ],
        out_specs=pl.BlockSpec((BT, K), lambda bt, dt: (bt, 0)),
        out_shape=jax.ShapeDtypeStruct((B, K), jnp.int32),
        scratch_shapes=[
            pltpu.VMEM((BT, E), jnp.float32),
            pltpu.VMEM((BT, K), jnp.float32),
            pltpu.VMEM((BT, K), jnp.float32),
        ],
        compiler_params=pltpu.CompilerParams(
            dimension_semantics=("arbitrary", "arbitrary")),
    )(obs, w, db)


def _gather_body(db_hbm, idx_hbm, out_hbm, idx_v, rows_v, sem):
    wid = lax.axis_index("s") * _NC + lax.axis_index("c")
    base = wid * _BPW
    pltpu.sync_copy(idx_hbm.at[pl.ds(base, _BPW)], idx_v)
    pltpu.async_copy(db_hbm.at[idx_v], rows_v, sem).wait()
    pltpu.sync_copy(rows_v, out_hbm.at[pl.ds(base, _BPW)])


@functools.lru_cache(maxsize=1)
def _gather_call():
    return pl.kernel(
        _gather_body,
        mesh=plsc.VectorSubcoreMesh(core_axis_name="c", subcore_axis_name="s"),
        out_type=jax.ShapeDtypeStruct((B * K, E), jnp.float32),
        scratch_types=[
            pltpu.VMEM((_BPW,), jnp.int32),
            pltpu.VMEM((_BPW, E), jnp.float32),
            pltpu.SemaphoreType.DMA,
        ],
    )


def kernel(obs, W_obs, db_embeddings, top_k):
    del top_k
    idx = _topk_call(obs, W_obs, db_embeddings)
    rows = _gather_call()(db_embeddings, idx.reshape(B * K))
    memory = rows.reshape(B, K * E)
    return memory, idx
